# Initial kernel scaffold; baseline (speedup 1.0000x reference)
import functools

import jax
import jax.numpy as jnp
from jax import lax
from jax.experimental import pallas as pl
from jax.experimental.pallas import tpu as pltpu
from jax.experimental.pallas import tpu_sc as plsc

B = 16384
F = 26
V = 100000
D = 16
ND = 13

NC = 2
NS = 16
NW = NC * NS

ROWS = B * F
ROWS_PER_W = ROWS // NW
CHUNK = 128
GROUP = 8
GROUP_ROWS = CHUNK * GROUP
N_GROUPS = ROWS_PER_W // GROUP_ROWS


def _gather_body(tbl_hbm, idx_hbm, out_hbm, idx_v, rows_v, sem):
    wid = lax.axis_index("s") * NC + lax.axis_index("c")
    row_base = wid * ROWS_PER_W
    pltpu.sync_copy(idx_hbm.at[pl.ds(row_base, ROWS_PER_W)], idx_v)

    def group(g, carry):
        base = g * GROUP_ROWS
        copies = []
        for j in range(GROUP):
            off = base + j * CHUNK
            copies.append(
                pltpu.async_copy(
                    tbl_hbm.at[idx_v.at[pl.ds(off, CHUNK)]],
                    rows_v.at[pl.ds(j * CHUNK, CHUNK)],
                    sem,
                )
            )
        for c in copies:
            c.wait()
        pltpu.sync_copy(rows_v, out_hbm.at[pl.ds(row_base + base, GROUP_ROWS)])
        return carry

    lax.fori_loop(0, N_GROUPS, group, 0)


def _sc_gather(flat_tables, flat_idx):
    mesh = plsc.VectorSubcoreMesh(core_axis_name="c", subcore_axis_name="s")
    return pl.kernel(
        _gather_body,
        out_type=jax.ShapeDtypeStruct((ROWS, D), jnp.float32),
        mesh=mesh,
        scratch_types=[
            pltpu.VMEM((ROWS_PER_W,), jnp.int32),
            pltpu.VMEM((GROUP_ROWS, D), jnp.float32),
            pltpu.SemaphoreType.DMA,
        ],
    )(flat_tables, flat_idx)


def _mlp_body(emb_ref, dense_ref, wbot_ref, bbot_ref, w1a_ref, w1b_ref,
              b1_ref, w2_ref, b2_ref, w3_ref, b3_ref, w4_ref, b4_ref,
              out_ref):
    f32 = jnp.float32
    demb = jnp.dot(dense_ref[...], wbot_ref[...], preferred_element_type=f32)
    demb = demb + bbot_ref[...]
    h = jnp.dot(emb_ref[...], w1a_ref[...], preferred_element_type=f32)
    h = h + jnp.dot(demb, w1b_ref[...], preferred_element_type=f32)
    h = jnp.maximum(h + b1_ref[...], 0.0)
    h = jnp.maximum(jnp.dot(h, w2_ref[...], preferred_element_type=f32) + b2_ref[...], 0.0)
    h = jnp.maximum(jnp.dot(h, w3_ref[...], preferred_element_type=f32) + b3_ref[...], 0.0)
    o = jnp.dot(h, w4_ref[...], preferred_element_type=f32) + b4_ref[...]
    out_ref[...] = jax.nn.sigmoid(o)


_BB = 2048


def _mlp(emb, dense, wbot, bbot, w1a, w1b, b1, w2, b2, w3, b3, w4, b4):
    full = lambda shape: pl.BlockSpec(shape, lambda i: (0, 0))
    return pl.pallas_call(
        _mlp_body,
        grid=(B // _BB,),
        in_specs=[
            pl.BlockSpec((_BB, F * D), lambda i: (i, 0)),
            pl.BlockSpec((_BB, ND), lambda i: (i, 0)),
            full((ND, D)),
            full((1, D)),
            full((F * D, 256)),
            full((D, 256)),
            full((1, 256)),
            full((256, 128)),
            full((1, 128)),
            full((128, 64)),
            full((1, 64)),
            full((64, 1)),
            full((1, 1)),
        ],
        out_specs=pl.BlockSpec((_BB, 1), lambda i: (i, 0)),
        out_shape=jax.ShapeDtypeStruct((B, 1), jnp.float32),
    )(emb, dense, wbot, bbot, w1a, w1b, b1, w2, b2, w3, b3, w4, b4)


def kernel(inputs_sparse, inputs_dense, tables, W_bot, b_bot,
           W1, b1, W2, b2, W3, b3, W4, b4):
    idx = inputs_sparse.astype(jnp.int32)
    idx = idx + (jnp.arange(F, dtype=jnp.int32) * V)[:, None]
    flat_idx = idx.T.reshape(-1)
    flat_tables = tables.reshape(F * V, D)
    emb = _sc_gather(flat_tables, flat_idx)
    emb = emb.reshape(B, F * D)
    out = _mlp(
        emb, inputs_dense, W_bot, b_bot.reshape(1, D),
        W1[: F * D], W1[F * D:], b1.reshape(1, 256),
        W2, b2.reshape(1, 128), W3, b3.reshape(1, 64),
        W4, b4.reshape(1, 1),
    )
    return out.reshape(-1)


# speedup vs baseline: 7.7720x; 7.7720x over previous
import functools

import jax
import jax.numpy as jnp
from jax import lax
from jax.experimental import pallas as pl
from jax.experimental.pallas import tpu as pltpu
from jax.experimental.pallas import tpu_sc as plsc

B = 16384
F = 26
V = 100000
D = 16
ND = 13

NC = 2
NS = 16
NW = NC * NS

ROWS = B * F
ROWS_PER_W = ROWS // NW
CHUNK = 128
GROUP = 8
GROUP_ROWS = CHUNK * GROUP
N_GROUPS = ROWS_PER_W // GROUP_ROWS


def _gather_body(tbl_hbm, idx_hbm, out_hbm, idx_v, rows_v, sem):
    wid = lax.axis_index("s") * NC + lax.axis_index("c")
    row_base = wid * ROWS_PER_W
    pltpu.sync_copy(idx_hbm.at[pl.ds(row_base, ROWS_PER_W)], idx_v)

    def group(g, carry):
        base = g * GROUP_ROWS
        copies = []
        for j in range(GROUP):
            off = base + j * CHUNK
            copies.append(
                pltpu.async_copy(
                    tbl_hbm.at[idx_v.at[pl.ds(off, CHUNK)]],
                    rows_v.at[pl.ds(j * CHUNK, CHUNK)],
                    sem,
                )
            )
        for c in copies:
            c.wait()
        pltpu.sync_copy(rows_v, out_hbm.at[pl.ds(row_base + base, GROUP_ROWS)])
        return carry

    lax.fori_loop(0, N_GROUPS, group, 0)


def _sc_gather(flat_tables, flat_idx):
    mesh = plsc.VectorSubcoreMesh(core_axis_name="c", subcore_axis_name="s")
    return pl.kernel(
        _gather_body,
        out_type=jax.ShapeDtypeStruct((ROWS, D), jnp.float32),
        mesh=mesh,
        scratch_types=[
            pltpu.VMEM((ROWS_PER_W,), jnp.int32),
            pltpu.VMEM((GROUP_ROWS, D), jnp.float32),
            pltpu.SemaphoreType.DMA,
        ],
        compiler_params=pltpu.CompilerParams(use_tc_tiling_on_sc=False),
    )(flat_tables, flat_idx)


def _mlp_body(emb_ref, dense_ref, wbot_ref, bbot_ref, w1a_ref, w1b_ref,
              b1_ref, w2_ref, b2_ref, w3_ref, b3_ref, w4_ref, b4_ref,
              out_ref):
    f32 = jnp.float32
    demb = jnp.dot(dense_ref[...], wbot_ref[...], preferred_element_type=f32)
    demb = demb + bbot_ref[...]
    h = jnp.dot(emb_ref[...], w1a_ref[...], preferred_element_type=f32)
    h = h + jnp.dot(demb, w1b_ref[...], preferred_element_type=f32)
    h = jnp.maximum(h + b1_ref[...], 0.0)
    h = jnp.maximum(jnp.dot(h, w2_ref[...], preferred_element_type=f32) + b2_ref[...], 0.0)
    h = jnp.maximum(jnp.dot(h, w3_ref[...], preferred_element_type=f32) + b3_ref[...], 0.0)
    o = jnp.dot(h, w4_ref[...], preferred_element_type=f32) + b4_ref[...]
    out_ref[...] = jax.nn.sigmoid(o)


_BB = 2048


def _mlp(emb, dense, wbot, bbot, w1a, w1b, b1, w2, b2, w3, b3, w4, b4):
    full = lambda shape: pl.BlockSpec(shape, lambda i: (0, 0))
    return pl.pallas_call(
        _mlp_body,
        grid=(B // _BB,),
        in_specs=[
            pl.BlockSpec((_BB, F * D), lambda i: (i, 0)),
            pl.BlockSpec((_BB, ND), lambda i: (i, 0)),
            full((ND, D)),
            full((1, D)),
            full((F * D, 256)),
            full((D, 256)),
            full((1, 256)),
            full((256, 128)),
            full((1, 128)),
            full((128, 64)),
            full((1, 64)),
            full((64, 1)),
            full((1, 1)),
        ],
        out_specs=pl.BlockSpec((_BB, 1), lambda i: (i, 0)),
        out_shape=jax.ShapeDtypeStruct((B, 1), jnp.float32),
    )(emb, dense, wbot, bbot, w1a, w1b, b1, w2, b2, w3, b3, w4, b4)


def kernel(inputs_sparse, inputs_dense, tables, W_bot, b_bot,
           W1, b1, W2, b2, W3, b3, W4, b4):
    idx = inputs_sparse.astype(jnp.int32)
    idx = idx + (jnp.arange(F, dtype=jnp.int32) * V)[:, None]
    flat_idx = idx.T.reshape(-1)
    flat_tables = tables.reshape(F * V, D)
    emb = _sc_gather(flat_tables, flat_idx)
    emb = emb.reshape(B, F * D)
    out = _mlp(
        emb, inputs_dense, W_bot, b_bot.reshape(1, D),
        W1[: F * D], W1[F * D:], b1.reshape(1, 256),
        W2, b2.reshape(1, 128), W3, b3.reshape(1, 64),
        W4, b4.reshape(1, 1),
    )
    return out.reshape(-1)
